# baseline (speedup 1.0000x reference)
import functools
import math

import jax
import jax.numpy as jnp
from jax import lax
from jax.experimental import pallas as pl
from jax.experimental.pallas import tpu as pltpu
from jax.experimental.pallas import tpu_sc as plsc

D_MODEL = 64
SCALE = math.sqrt(D_MODEL)

_NC = 2
_NS = 16
_NW = _NC * _NS
_LANES = 16


@functools.lru_cache(maxsize=None)
def _build(B: int, V: int, D: int):
    assert B % _NW == 0
    per_w = B // _NW
    chunk = 640
    assert per_w % chunk == 0
    nchunk = per_w // chunk
    vregs_per_row = D // _LANES

    mesh = plsc.VectorSubcoreMesh(core_axis_name="c", subcore_axis_name="s")

    @functools.partial(
        pl.kernel,
        mesh=mesh,
        compiler_params=pltpu.CompilerParams(use_tc_tiling_on_sc=False),
        out_type=jax.ShapeDtypeStruct((B, D), jnp.float32),
        scratch_types=[
            pltpu.VMEM((per_w,), jnp.int32),
            pltpu.VMEM((2, chunk, D), jnp.float32),
            pltpu.SemaphoreType.DMA,
            pltpu.SemaphoreType.DMA,
            pltpu.SemaphoreType.DMA,
            pltpu.SemaphoreType.DMA,
        ],
    )
    def emb(x_hbm, lut_hbm, out_hbm, idx_v, rows_v, g0, g1, w0, w1):
        wid = lax.axis_index("s") * _NC + lax.axis_index("c")
        base = wid * per_w
        gsem = (g0, g1)
        wsem = (w0, w1)

        pltpu.sync_copy(x_hbm.at[pl.ds(base, per_w)], idx_v)

        nsub = 4
        sub = chunk // nsub

        def start_gather(g):
            b = g % 2
            return [
                pltpu.async_copy(
                    lut_hbm.at[idx_v.at[pl.ds(g * chunk + j * sub, sub)]],
                    rows_v.at[b].at[pl.ds(j * sub, sub)],
                    gsem[b],
                )
                for j in range(nsub)
            ]

        def scale_chunk(b):
            def scale_body(r, c2):
                for c in range(vregs_per_row):
                    sl = pl.ds(c * _LANES, _LANES)
                    rows_v[b, r, sl] = rows_v[b, r, sl] * SCALE
                return c2

            lax.fori_loop(0, chunk, scale_body, 0)

        def start_write(g):
            b = g % 2
            return pltpu.async_copy(
                rows_v.at[b],
                out_hbm.at[pl.ds(base + g * chunk, chunk)],
                wsem[b],
            )

        gathers = {0: start_gather(0)}
        writes = {}
        for g in range(nchunk):
            if g + 1 < nchunk:
                if g - 1 >= 0:
                    writes.pop(g - 1).wait()
                gathers[g + 1] = start_gather(g + 1)
            for d in gathers.pop(g):
                d.wait()
            scale_chunk(g % 2)
            writes[g] = start_write(g)
        for g in sorted(writes):
            writes.pop(g).wait()

    return emb


def kernel(x, lut):
    orig_shape = x.shape
    xf = x.reshape(-1).astype(jnp.int32)
    V, D = lut.shape
    out = _build(xf.shape[0], V, D)(xf, lut)
    return out.reshape(*orig_shape, D)


# speedup vs baseline: 1.0009x; 1.0009x over previous
import functools
import math

import jax
import jax.numpy as jnp
from jax import lax
from jax.experimental import pallas as pl
from jax.experimental.pallas import tpu as pltpu
from jax.experimental.pallas import tpu_sc as plsc

D_MODEL = 64
SCALE = math.sqrt(D_MODEL)

_NC = 2
_NS = 16
_NW = _NC * _NS
_LANES = 16


@functools.lru_cache(maxsize=None)
def _build(R: int, S: int, V: int, D: int):
    assert R % _NW == 0
    rx = R // _NW
    gx = 8
    assert rx % gx == 0
    nchunk = rx // gx
    chunk = gx * S
    vregs_per_row = D // _LANES

    mesh = plsc.VectorSubcoreMesh(core_axis_name="c", subcore_axis_name="s")

    @functools.partial(
        pl.kernel,
        mesh=mesh,
        compiler_params=pltpu.CompilerParams(use_tc_tiling_on_sc=False),
        out_type=jax.ShapeDtypeStruct((R * S, D), jnp.float32),
        scratch_types=[
            pltpu.VMEM((rx, S), jnp.int32),
            pltpu.VMEM((2, chunk, D), jnp.float32),
            pltpu.SemaphoreType.DMA,
            pltpu.SemaphoreType.DMA,
            pltpu.SemaphoreType.DMA,
            pltpu.SemaphoreType.DMA,
        ],
    )
    def emb(x_hbm, lut_hbm, out_hbm, idx_v, rows_v, g0, g1, w0, w1):
        wid = lax.axis_index("s") * _NC + lax.axis_index("c")
        gsem = (g0, g1)
        wsem = (w0, w1)

        pltpu.sync_copy(x_hbm.at[pl.ds(wid * rx, rx)], idx_v)

        def start_gather(g):
            b = g % 2
            return [
                pltpu.async_copy(
                    lut_hbm.at[idx_v.at[g * gx + j]],
                    rows_v.at[b].at[pl.ds(j * S, S)],
                    gsem[b],
                )
                for j in range(gx)
            ]

        def scale_chunk(b):
            def scale_body(r, c2):
                for c in range(vregs_per_row):
                    sl = pl.ds(c * _LANES, _LANES)
                    rows_v[b, r, sl] = rows_v[b, r, sl] * SCALE
                return c2

            lax.fori_loop(0, chunk, scale_body, 0)

        def start_write(g):
            b = g % 2
            return pltpu.async_copy(
                rows_v.at[b],
                out_hbm.at[pl.ds(wid * rx * S + g * chunk, chunk)],
                wsem[b],
            )

        gathers = {0: start_gather(0)}
        writes = {}
        for g in range(nchunk):
            if g + 1 < nchunk:
                if g - 1 >= 0:
                    writes.pop(g - 1).wait()
                gathers[g + 1] = start_gather(g + 1)
            for d in gathers.pop(g):
                d.wait()
            scale_chunk(g % 2)
            writes[g] = start_write(g)
        for g in sorted(writes):
            writes.pop(g).wait()

    return emb


def kernel(x, lut):
    R, S = x.shape
    V, D = lut.shape
    out = _build(R, S, V, D)(x, lut)
    return out.reshape(R, S, D)


# speedup vs baseline: 1.1271x; 1.1261x over previous
import functools
import math

import jax
import jax.numpy as jnp
from jax import lax
from jax.experimental import pallas as pl
from jax.experimental.pallas import tpu as pltpu
from jax.experimental.pallas import tpu_sc as plsc

D_MODEL = 64
SCALE = math.sqrt(D_MODEL)

_NC = 2
_NS = 16
_NW = _NC * _NS
_LANES = 16


@functools.lru_cache(maxsize=None)
def _build(R: int, S: int, V: int, D: int):
    assert R % _NW == 0
    rx = R // _NW
    gx = 8
    assert rx % gx == 0
    nchunk = rx // gx
    vregs_per_row = D // _LANES

    mesh = plsc.VectorSubcoreMesh(core_axis_name="c", subcore_axis_name="s")

    @functools.partial(
        pl.kernel,
        mesh=mesh,
        compiler_params=pltpu.CompilerParams(use_tc_tiling_on_sc=False),
        out_type=jax.ShapeDtypeStruct((R, 56, 2 * D), jnp.float32),
        scratch_types=[
            pltpu.VMEM((rx, S), jnp.int32),
            pltpu.VMEM((2, gx, S, D), jnp.float32),
            pltpu.SemaphoreType.DMA,
            pltpu.SemaphoreType.DMA,
            pltpu.SemaphoreType.DMA,
            pltpu.SemaphoreType.DMA,
        ],
    )
    def emb(x_hbm, lut_hbm, out_hbm, idx_v, rows_v, g0, g1, w0, w1):
        wid = lax.axis_index("s") * _NC + lax.axis_index("c")
        gsem = (g0, g1)
        wsem = (w0, w1)

        pltpu.sync_copy(x_hbm.at[pl.ds(wid * rx, rx)], idx_v)

        def start_gather(g):
            b = g % 2
            return [
                pltpu.async_copy(
                    lut_hbm.at[idx_v.at[g * gx + j]],
                    rows_v.at[b].at[j],
                    gsem[b],
                )
                for j in range(gx)
            ]

        def scale_chunk(b):
            def scale_body(r, c2):
                for j in range(gx):
                    for c in range(vregs_per_row):
                        sl = pl.ds(c * _LANES, _LANES)
                        rows_v[b, j, r, sl] = rows_v[b, j, r, sl] * SCALE
                return c2

            lax.fori_loop(0, S, scale_body, 0)

        def start_write(g):
            b = g % 2
            return pltpu.async_copy(
                rows_v.at[b],
                out_hbm.at[pl.ds(wid * rx + g * gx, gx)].at[
                    :, pl.ds(0, S), pl.ds(0, D)],
                wsem[b],
            )

        gathers = {0: start_gather(0)}
        writes = {}
        for g in range(nchunk):
            if g + 1 < nchunk:
                if g - 1 >= 0:
                    writes.pop(g - 1).wait()
                gathers[g + 1] = start_gather(g + 1)
            for d in gathers.pop(g):
                d.wait()
            scale_chunk(g % 2)
            writes[g] = start_write(g)
        for g in sorted(writes):
            writes.pop(g).wait()

    return emb


def kernel(x, lut):
    R, S = x.shape
    V, D = lut.shape
    out = _build(R, S, V, D)(x, lut)
    return out[:, :S, :D]


# speedup vs baseline: 1.1985x; 1.0633x over previous
import functools
import math

import jax
import jax.numpy as jnp
from jax import lax
from jax.experimental import pallas as pl
from jax.experimental.pallas import tpu as pltpu
from jax.experimental.pallas import tpu_sc as plsc

D_MODEL = 64
SCALE = math.sqrt(D_MODEL)

_NC = 2
_NS = 16
_NW = _NC * _NS
_LANES = 16


@functools.lru_cache(maxsize=None)
def _build(R: int, S: int, V: int, D: int):
    assert R % _NW == 0
    rx = R // _NW
    gx = 8
    assert rx % gx == 0
    nchunk = rx // gx
    vregs_per_row = D // _LANES

    mesh = plsc.VectorSubcoreMesh(core_axis_name="c", subcore_axis_name="s")

    @functools.partial(
        pl.kernel,
        mesh=mesh,
        compiler_params=pltpu.CompilerParams(use_tc_tiling_on_sc=False),
        out_type=jax.ShapeDtypeStruct((R, 56, 2 * D), jnp.float32),
        scratch_types=[
            pltpu.VMEM((rx, S), jnp.int32),
            pltpu.VMEM((2, gx, S, 2 * D), jnp.float32),
            pltpu.SemaphoreType.DMA,
            pltpu.SemaphoreType.DMA,
            pltpu.SemaphoreType.DMA,
            pltpu.SemaphoreType.DMA,
        ],
    )
    def emb(x_hbm, lut_hbm, out_hbm, idx_v, rows_v, g0, g1, w0, w1):
        wid = lax.axis_index("s") * _NC + lax.axis_index("c")
        gsem = (g0, g1)
        wsem = (w0, w1)

        pltpu.sync_copy(x_hbm.at[pl.ds(wid * rx, rx)], idx_v)

        def start_gather(g):
            b = g % 2
            return [
                pltpu.async_copy(
                    lut_hbm.at[idx_v.at[g * gx + j]],
                    rows_v.at[b].at[j],
                    gsem[b],
                )
                for j in range(gx)
            ]

        def scale_chunk(b):
            def scale_body(r, c2):
                for j in range(gx):
                    for c in range(vregs_per_row):
                        sl = pl.ds(c * _LANES, _LANES)
                        rows_v[b, j, r, sl] = rows_v[b, j, r, sl] * SCALE
                return c2

            lax.fori_loop(0, S, scale_body, 0)

        def start_write(g):
            b = g % 2
            return pltpu.async_copy(
                rows_v.at[b].at[:, :, pl.ds(0, D)],
                out_hbm.at[pl.ds(wid * rx + g * gx, gx)].at[
                    :, pl.ds(0, S), pl.ds(0, D)],
                wsem[b],
            )

        gathers = {0: start_gather(0)}
        writes = {}
        for g in range(nchunk):
            if g + 1 < nchunk:
                if g - 1 >= 0:
                    writes.pop(g - 1).wait()
                gathers[g + 1] = start_gather(g + 1)
            for d in gathers.pop(g):
                d.wait()
            scale_chunk(g % 2)
            writes[g] = start_write(g)
        for g in sorted(writes):
            writes.pop(g).wait()

    return emb


def kernel(x, lut):
    R, S = x.shape
    V, D = lut.shape
    lut_p = jnp.pad(lut, ((0, 0), (0, D)))
    out = _build(R, S, V, D)(x, lut_p)
    return out[:, :S, :D]
